# baseline (speedup 1.0000x reference)
import functools

import jax
import jax.numpy as jnp
from jax import lax
from jax.experimental import pallas as pl
from jax.experimental.pallas import tpu as pltpu
from jax.experimental.pallas import tpu_sc as plsc

_B = 16384
_V_USER, _V_SMALL = 1000000, 1000
_D_USER, _D_COUNTRY, _D_DEVICE, _D_DENSE = 32, 16, 16, 128
_TOTAL = _D_USER + _D_COUNTRY + _D_DEVICE + _D_DENSE
_OUT_D = 128
_EPS = 1.1920928955078125e-07

_NC, _NS = 2, 16
_BPS = _B // _NS
_STAGE = 65536
_STAGE_TAIL = _V_USER - 15 * _STAGE


def _sc_user_body(uid_hbm, eu_t, out_u, uidx_v, dst_v, row_a, row_b,
                  sem, sem_st):
    c = lax.axis_index("c")
    s = lax.axis_index("s")

    def stage_copies(jl, row_sh):
        j = c * _NS + jl
        main = pltpu.make_async_copy(
            eu_t.at[pl.ds(j, 1), pl.ds(s * _STAGE, _STAGE)],
            row_sh.at[pl.ds(0, 1), pl.ds(s * _STAGE, _STAGE)],
            sem_st)
        tail = pltpu.make_async_copy(
            eu_t.at[pl.ds(j, 1), pl.ds(15 * _STAGE, _STAGE_TAIL)],
            row_sh.at[pl.ds(0, 1), pl.ds(15 * _STAGE, _STAGE_TAIL)],
            sem_st)
        return main, tail

    def stage_start(jl, row_sh):
        main, tail = stage_copies(jl, row_sh)

        @pl.when(s < 15)
        def _():
            main.start()

        @pl.when(s == 15)
        def _():
            tail.start()

    def stage_wait(jl, row_sh):
        main, tail = stage_copies(jl, row_sh)

        @pl.when(s < 15)
        def _():
            main.wait()

        @pl.when(s == 15)
        def _():
            tail.wait()

    stage_start(0, row_a)
    pltpu.sync_copy(uid_hbm.at[pl.ds(s * _BPS, _BPS)], uidx_v)
    stage_wait(0, row_a)
    plsc.subcore_barrier()

    def half_step(jl, cur_row, nxt_row):
        j = c * _NS + jl

        @pl.when(jl < _NS - 1)
        def _():
            stage_start(jl + 1, nxt_row)

        pltpu.async_copy(cur_row.at[0].at[uidx_v], dst_v.at[0], sem).wait()
        pltpu.sync_copy(dst_v, out_u.at[pl.ds(j, 1), pl.ds(s * _BPS, _BPS)])

        @pl.when(jl < _NS - 1)
        def _():
            stage_wait(jl + 1, nxt_row)

        plsc.subcore_barrier()

    def step(p, carry):
        half_step(2 * p, row_a, row_b)
        half_step(2 * p + 1, row_b, row_a)
        return carry

    lax.fori_loop(0, _NS // 2, step, 0)


def _sc_user_gather(user_id, emb_user_t):
    return pl.kernel(
        _sc_user_body,
        out_type=jax.ShapeDtypeStruct((_D_USER, _B), jnp.float32),
        mesh=plsc.VectorSubcoreMesh(core_axis_name="c", subcore_axis_name="s"),
        compiler_params=pltpu.CompilerParams(use_tc_tiling_on_sc=True),
        scratch_types=[
            pltpu.VMEM((_BPS,), jnp.int32),
            pltpu.VMEM((1, _BPS), jnp.float32),
            pltpu.VMEM_SHARED((1, _V_USER), jnp.float32),
            pltpu.VMEM_SHARED((1, _V_USER), jnp.float32),
            pltpu.SemaphoreType.DMA,
            pltpu.SemaphoreType.DMA,
        ],
    )(user_id, emb_user_t)


def _sc_small_body(cid_hbm, did_hbm, ec_t, ed_t, out_c, out_d,
                   cidx_v, didx_v, ecT_v, edT_v, cbuf_v, dbuf_v):
    c = lax.axis_index("c")
    s = lax.axis_index("s")
    half = _BPS // 2
    base = s * _BPS + c * half
    pltpu.sync_copy(cid_hbm.at[pl.ds(base, half)], cidx_v)
    pltpu.sync_copy(did_hbm.at[pl.ds(base, half)], didx_v)
    pltpu.sync_copy(ec_t, ecT_v)
    pltpu.sync_copy(ed_t, edT_v)

    def small_row(jj, _):
        rows = jnp.zeros((16,), jnp.int32) + jj

        def group(g, _):
            cbuf_v[jj, pl.ds(g * 16, 16)] = plsc.load_gather(
                ecT_v, [rows, cidx_v[pl.ds(g * 16, 16)]])
            dbuf_v[jj, pl.ds(g * 16, 16)] = plsc.load_gather(
                edT_v, [rows, didx_v[pl.ds(g * 16, 16)]])
            return 0

        return lax.fori_loop(0, half // 16, group, 0)

    lax.fori_loop(0, _D_COUNTRY, small_row, 0)
    pltpu.sync_copy(cbuf_v, out_c.at[pl.ds(0, _D_COUNTRY), pl.ds(base, half)])
    pltpu.sync_copy(dbuf_v, out_d.at[pl.ds(0, _D_DEVICE), pl.ds(base, half)])


def _sc_small_gather(country, device, emb_country_t, emb_device_t):
    half = _BPS // 2
    return pl.kernel(
        _sc_small_body,
        out_type=[
            jax.ShapeDtypeStruct((_D_COUNTRY, _B), jnp.float32),
            jax.ShapeDtypeStruct((_D_DEVICE, _B), jnp.float32),
        ],
        mesh=plsc.VectorSubcoreMesh(core_axis_name="c", subcore_axis_name="s"),
        compiler_params=pltpu.CompilerParams(use_tc_tiling_on_sc=True,
                                             needs_layout_passes=False),
        scratch_types=[
            pltpu.VMEM((half,), jnp.int32),
            pltpu.VMEM((half,), jnp.int32),
            pltpu.VMEM((_D_COUNTRY, _V_SMALL), jnp.float32),
            pltpu.VMEM((_D_DEVICE, _V_SMALL), jnp.float32),
            pltpu.VMEM((_D_COUNTRY, half), jnp.float32),
            pltpu.VMEM((_D_DEVICE, half), jnp.float32),
        ],
    )(country, device, emb_country_t, emb_device_t)


def _tc_body(ut_ref, ct_ref, dt_ref, x_ref, rw_ref, w_ref, b_ref, out_ref):
    ut = ut_ref[...]
    ct = ct_ref[...]
    dt = dt_ref[...]
    x = x_ref[...]
    dnums_t = (((0,), (0,)), ((), ()))
    ones_u = jnp.ones((_D_USER, 1), jnp.float32)
    ones_s = jnp.ones((_D_COUNTRY, 1), jnp.float32)
    ssq = (lax.dot_general(ut * ut, ones_u, dnums_t,
                           preferred_element_type=jnp.float32)
           + lax.dot_general(ct * ct, ones_s, dnums_t,
                             preferred_element_type=jnp.float32)
           + lax.dot_general(dt * dt, ones_s, dnums_t,
                             preferred_element_type=jnp.float32)
           + jnp.sum(x * x, axis=1, keepdims=True))
    scale = lax.rsqrt(ssq * (1.0 / _TOTAL) + _EPS)
    ws = w_ref[...] * rw_ref[...]
    s0, s1, s2 = _D_USER, _D_USER + _D_COUNTRY, _D_USER + _D_COUNTRY + _D_DEVICE
    acc = lax.dot_general(ut, ws[0:s0], dnums_t,
                          preferred_element_type=jnp.float32)
    acc += lax.dot_general(ct, ws[s0:s1], dnums_t,
                           preferred_element_type=jnp.float32)
    acc += lax.dot_general(dt, ws[s1:s2], dnums_t,
                           preferred_element_type=jnp.float32)
    acc += jnp.dot(x, ws[s2:_TOTAL], preferred_element_type=jnp.float32)
    out_ref[...] = acc * scale + b_ref[...]


def _tc_norm_linear(e_user_t, e_country_t, e_device_t, dense_profile,
                    rms_weight, W, b, blk=1024):
    grid = _B // blk
    rw = rms_weight.reshape(_TOTAL, 1)
    b2 = b.reshape(1, _OUT_D)
    return pl.pallas_call(
        _tc_body,
        grid=(grid,),
        in_specs=[
            pl.BlockSpec((_D_USER, blk), lambda i: (0, i)),
            pl.BlockSpec((_D_COUNTRY, blk), lambda i: (0, i)),
            pl.BlockSpec((_D_DEVICE, blk), lambda i: (0, i)),
            pl.BlockSpec((blk, _D_DENSE), lambda i: (i, 0)),
            pl.BlockSpec((_TOTAL, 1), lambda i: (0, 0)),
            pl.BlockSpec((_TOTAL, _OUT_D), lambda i: (0, 0)),
            pl.BlockSpec((1, _OUT_D), lambda i: (0, 0)),
        ],
        out_specs=pl.BlockSpec((blk, _OUT_D), lambda i: (i, 0)),
        out_shape=jax.ShapeDtypeStruct((_B, _OUT_D), jnp.float32),
    )(e_user_t, e_country_t, e_device_t, dense_profile, rw, W, b2)


def kernel(user_id, country, device, dense_profile, emb_user, emb_country,
           emb_device, rms_weight, W, b):
    e_user_t = _sc_user_gather(user_id.astype(jnp.int32), emb_user.T)
    e_country_t, e_device_t = _sc_small_gather(country, device,
                                               emb_country.T, emb_device.T)
    return _tc_norm_linear(e_user_t, e_country_t, e_device_t, dense_profile,
                           rms_weight, W, b)


# speedup vs baseline: 1.0876x; 1.0876x over previous
import functools

import jax
import jax.numpy as jnp
from jax import lax
from jax.experimental import pallas as pl
from jax.experimental.pallas import tpu as pltpu
from jax.experimental.pallas import tpu_sc as plsc

_B = 16384
_V_USER, _V_SMALL = 1000000, 1000
_D_USER, _D_COUNTRY, _D_DEVICE, _D_DENSE = 32, 16, 16, 128
_TOTAL = _D_USER + _D_COUNTRY + _D_DEVICE + _D_DENSE
_OUT_D = 128
_EPS = 1.1920928955078125e-07

_NC, _NS = 2, 16
_BPS = _B // _NS
_STAGE = 65536
_STAGE_TAIL = _V_USER - 15 * _STAGE


def _sc_user_body(uid_hbm, eu_t, out_u, uidx_v, dst_v, row_a, row_b,
                  sem, sem_st):
    c = lax.axis_index("c")
    s = lax.axis_index("s")

    def stage_copies(jl, row_sh):
        j = c * _NS + jl
        main = pltpu.make_async_copy(
            eu_t.at[pl.ds(j, 1), pl.ds(s * _STAGE, _STAGE)],
            row_sh.at[pl.ds(0, 1), pl.ds(s * _STAGE, _STAGE)],
            sem_st)
        tail = pltpu.make_async_copy(
            eu_t.at[pl.ds(j, 1), pl.ds(15 * _STAGE, _STAGE_TAIL)],
            row_sh.at[pl.ds(0, 1), pl.ds(15 * _STAGE, _STAGE_TAIL)],
            sem_st)
        return main, tail

    def stage_start(jl, row_sh):
        main, tail = stage_copies(jl, row_sh)

        @pl.when(s < 15)
        def _():
            main.start()

        @pl.when(s == 15)
        def _():
            tail.start()

    def stage_wait(jl, row_sh):
        main, tail = stage_copies(jl, row_sh)

        @pl.when(s < 15)
        def _():
            main.wait()

        @pl.when(s == 15)
        def _():
            tail.wait()

    stage_start(0, row_a)
    pltpu.sync_copy(uid_hbm.at[pl.ds(s * _BPS, _BPS)], uidx_v)
    stage_wait(0, row_a)
    plsc.subcore_barrier()

    def half_step(jl, cur_row, nxt_row):
        j = c * _NS + jl

        @pl.when(jl < _NS - 1)
        def _():
            stage_start(jl + 1, nxt_row)

        pltpu.async_copy(cur_row.at[0].at[uidx_v], dst_v.at[0], sem).wait()
        pltpu.sync_copy(dst_v, out_u.at[pl.ds(j, 1), pl.ds(s * _BPS, _BPS)])

        @pl.when(jl < _NS - 1)
        def _():
            stage_wait(jl + 1, nxt_row)

        plsc.subcore_barrier()

    def step(p, carry):
        half_step(2 * p, row_a, row_b)
        half_step(2 * p + 1, row_b, row_a)
        return carry

    lax.fori_loop(0, _NS // 2, step, 0)


def _sc_user_gather(user_id, emb_user_t):
    return pl.kernel(
        _sc_user_body,
        out_type=jax.ShapeDtypeStruct((_D_USER, _B), jnp.float32),
        mesh=plsc.VectorSubcoreMesh(core_axis_name="c", subcore_axis_name="s"),
        compiler_params=pltpu.CompilerParams(use_tc_tiling_on_sc=True),
        scratch_types=[
            pltpu.VMEM((_BPS,), jnp.int32),
            pltpu.VMEM((1, _BPS), jnp.float32),
            pltpu.VMEM_SHARED((1, _V_USER), jnp.float32),
            pltpu.VMEM_SHARED((1, _V_USER), jnp.float32),
            pltpu.SemaphoreType.DMA,
            pltpu.SemaphoreType.DMA,
        ],
    )(user_id, emb_user_t)


_BPW = _B // (_NC * _NS)


def _sc_small_body(cid_hbm, did_hbm, ec_hbm, ed_hbm, out_c, out_d,
                   cidx_v, didx_v, crows_v, drows_v, sem_c, sem_d):
    wid = lax.axis_index("s") * _NC + lax.axis_index("c")
    base = wid * _BPW
    pltpu.sync_copy(cid_hbm.at[pl.ds(base, _BPW)], cidx_v)
    pltpu.sync_copy(did_hbm.at[pl.ds(base, _BPW)], didx_v)
    cp_c = pltpu.async_copy(ec_hbm.at[cidx_v], crows_v, sem_c)
    cp_d = pltpu.async_copy(ed_hbm.at[didx_v], drows_v, sem_d)
    cp_c.wait()
    cp_d.wait()
    pltpu.sync_copy(crows_v, out_c.at[pl.ds(base, _BPW)])
    pltpu.sync_copy(drows_v, out_d.at[pl.ds(base, _BPW)])


def _sc_small_gather(country, device, emb_country, emb_device):
    return pl.kernel(
        _sc_small_body,
        out_type=[
            jax.ShapeDtypeStruct((_B, _D_COUNTRY), jnp.float32),
            jax.ShapeDtypeStruct((_B, _D_DEVICE), jnp.float32),
        ],
        mesh=plsc.VectorSubcoreMesh(core_axis_name="c", subcore_axis_name="s"),
        compiler_params=pltpu.CompilerParams(use_tc_tiling_on_sc=False),
        scratch_types=[
            pltpu.VMEM((_BPW,), jnp.int32),
            pltpu.VMEM((_BPW,), jnp.int32),
            pltpu.VMEM((_BPW, _D_COUNTRY), jnp.float32),
            pltpu.VMEM((_BPW, _D_DEVICE), jnp.float32),
            pltpu.SemaphoreType.DMA,
            pltpu.SemaphoreType.DMA,
        ],
    )(country, device, emb_country, emb_device)


def _tc_dense_body(x_ref, rwx_ref, wx_ref, out_ref, ssq_ref):
    x = x_ref[...]
    ws = wx_ref[...] * rwx_ref[...]
    out_ref[...] = jnp.dot(x, ws, preferred_element_type=jnp.float32)
    ssq_ref[...] = jnp.sum(x * x, axis=1, keepdims=True)


def _tc_dense_partial(dense_profile, rms_weight, W, blk=1024):
    grid = _B // blk
    rwx = lax.slice(rms_weight, (_TOTAL - _D_DENSE,), (_TOTAL,)).reshape(
        _D_DENSE, 1)
    wx = lax.slice(W, (_TOTAL - _D_DENSE, 0), (_TOTAL, _OUT_D))
    return pl.pallas_call(
        _tc_dense_body,
        grid=(grid,),
        in_specs=[
            pl.BlockSpec((blk, _D_DENSE), lambda i: (i, 0)),
            pl.BlockSpec((_D_DENSE, 1), lambda i: (0, 0)),
            pl.BlockSpec((_D_DENSE, _OUT_D), lambda i: (0, 0)),
        ],
        out_specs=[
            pl.BlockSpec((blk, _OUT_D), lambda i: (i, 0)),
            pl.BlockSpec((blk, 1), lambda i: (i, 0)),
        ],
        out_shape=[
            jax.ShapeDtypeStruct((_B, _OUT_D), jnp.float32),
            jax.ShapeDtypeStruct((_B, 1), jnp.float32),
        ],
    )(dense_profile, rwx, wx)


_D_EMB = _D_USER + _D_COUNTRY + _D_DEVICE


def _tc_combine_body(ut_ref, c_ref, d_ref, part_ref, ssqx_ref, rwe_ref,
                     we_ref, b_ref, out_ref):
    ut = ut_ref[...]
    c = c_ref[...]
    d = d_ref[...]
    part = part_ref[...]
    dnums_t = (((0,), (0,)), ((), ()))
    ones_u = jnp.ones((_D_USER, 1), jnp.float32)
    ssq = (lax.dot_general(ut * ut, ones_u, dnums_t,
                           preferred_element_type=jnp.float32)
           + jnp.sum(c * c, axis=1, keepdims=True)
           + jnp.sum(d * d, axis=1, keepdims=True)
           + ssqx_ref[...])
    scale = lax.rsqrt(ssq * (1.0 / _TOTAL) + _EPS)
    ws = we_ref[...] * rwe_ref[...]
    s0, s1 = _D_USER, _D_USER + _D_COUNTRY
    acc = lax.dot_general(ut, ws[0:s0], dnums_t,
                          preferred_element_type=jnp.float32)
    acc += jnp.dot(c, ws[s0:s1], preferred_element_type=jnp.float32)
    acc += jnp.dot(d, ws[s1:_D_EMB], preferred_element_type=jnp.float32)
    acc += part
    out_ref[...] = acc * scale + b_ref[...]


def _tc_combine(e_user_t, e_country, e_device, partial, ssq_x, rms_weight,
                W, b, blk=1024):
    grid = _B // blk
    rwe = lax.slice(rms_weight, (0,), (_D_EMB,)).reshape(_D_EMB, 1)
    we = lax.slice(W, (0, 0), (_D_EMB, _OUT_D))
    b2 = b.reshape(1, _OUT_D)
    return pl.pallas_call(
        _tc_combine_body,
        grid=(grid,),
        in_specs=[
            pl.BlockSpec((_D_USER, blk), lambda i: (0, i)),
            pl.BlockSpec((blk, _D_COUNTRY), lambda i: (i, 0)),
            pl.BlockSpec((blk, _D_DEVICE), lambda i: (i, 0)),
            pl.BlockSpec((blk, _OUT_D), lambda i: (i, 0)),
            pl.BlockSpec((blk, 1), lambda i: (i, 0)),
            pl.BlockSpec((_D_EMB, 1), lambda i: (0, 0)),
            pl.BlockSpec((_D_EMB, _OUT_D), lambda i: (0, 0)),
            pl.BlockSpec((1, _OUT_D), lambda i: (0, 0)),
        ],
        out_specs=pl.BlockSpec((blk, _OUT_D), lambda i: (i, 0)),
        out_shape=jax.ShapeDtypeStruct((_B, _OUT_D), jnp.float32),
    )(e_user_t, e_country, e_device, partial, ssq_x, rwe, we, b2)


def kernel(user_id, country, device, dense_profile, emb_user, emb_country,
           emb_device, rms_weight, W, b):
    e_user_t = _sc_user_gather(user_id.astype(jnp.int32), emb_user.T)
    e_country, e_device = _sc_small_gather(country, device, emb_country,
                                           emb_device)
    partial, ssq_x = _tc_dense_partial(dense_profile, rms_weight, W)
    return _tc_combine(e_user_t, e_country, e_device, partial, ssq_x,
                       rms_weight, W, b)


# speedup vs baseline: 1.0997x; 1.0111x over previous
import functools

import jax
import jax.numpy as jnp
from jax import lax
from jax.experimental import pallas as pl
from jax.experimental.pallas import tpu as pltpu
from jax.experimental.pallas import tpu_sc as plsc

_B = 16384
_V_USER, _V_SMALL = 1000000, 1000
_D_USER, _D_COUNTRY, _D_DEVICE, _D_DENSE = 32, 16, 16, 128
_TOTAL = _D_USER + _D_COUNTRY + _D_DEVICE + _D_DENSE
_OUT_D = 128
_EPS = 1.1920928955078125e-07
_VPAD = 1024

_NC, _NS = 2, 16
_BPS = _B // _NS
_STAGE = 65536
_STAGE_TAIL = _V_USER - 15 * _STAGE


def _sc_user_body(uid_hbm, eu_t, out_u, uidx_v, dst_v, row_a, row_b,
                  sem, sem_st):
    c = lax.axis_index("c")
    s = lax.axis_index("s")

    def stage_copies(jl, row_sh):
        j = c * _NS + jl
        main = pltpu.make_async_copy(
            eu_t.at[pl.ds(j, 1), pl.ds(s * _STAGE, _STAGE)],
            row_sh.at[pl.ds(0, 1), pl.ds(s * _STAGE, _STAGE)],
            sem_st)
        tail = pltpu.make_async_copy(
            eu_t.at[pl.ds(j, 1), pl.ds(15 * _STAGE, _STAGE_TAIL)],
            row_sh.at[pl.ds(0, 1), pl.ds(15 * _STAGE, _STAGE_TAIL)],
            sem_st)
        return main, tail

    def stage_start(jl, row_sh):
        main, tail = stage_copies(jl, row_sh)

        @pl.when(s < 15)
        def _():
            main.start()

        @pl.when(s == 15)
        def _():
            tail.start()

    def stage_wait(jl, row_sh):
        main, tail = stage_copies(jl, row_sh)

        @pl.when(s < 15)
        def _():
            main.wait()

        @pl.when(s == 15)
        def _():
            tail.wait()

    stage_start(0, row_a)
    pltpu.sync_copy(uid_hbm.at[pl.ds(s * _BPS, _BPS)], uidx_v)
    stage_wait(0, row_a)
    plsc.subcore_barrier()

    def half_step(jl, cur_row, nxt_row):
        j = c * _NS + jl

        @pl.when(jl < _NS - 1)
        def _():
            stage_start(jl + 1, nxt_row)

        pltpu.async_copy(cur_row.at[0].at[uidx_v], dst_v.at[0], sem).wait()
        pltpu.sync_copy(dst_v, out_u.at[pl.ds(j, 1), pl.ds(s * _BPS, _BPS)])

        @pl.when(jl < _NS - 1)
        def _():
            stage_wait(jl + 1, nxt_row)

        plsc.subcore_barrier()

    def step(p, carry):
        half_step(2 * p, row_a, row_b)
        half_step(2 * p + 1, row_b, row_a)
        return carry

    lax.fori_loop(0, _NS // 2, step, 0)


def _sc_user_gather(user_id, emb_user_t):
    return pl.kernel(
        _sc_user_body,
        out_type=jax.ShapeDtypeStruct((_D_USER, _B), jnp.float32),
        mesh=plsc.VectorSubcoreMesh(core_axis_name="c", subcore_axis_name="s"),
        compiler_params=pltpu.CompilerParams(use_tc_tiling_on_sc=True),
        scratch_types=[
            pltpu.VMEM((_BPS,), jnp.int32),
            pltpu.VMEM((1, _BPS), jnp.float32),
            pltpu.VMEM_SHARED((1, _V_USER), jnp.float32),
            pltpu.VMEM_SHARED((1, _V_USER), jnp.float32),
            pltpu.SemaphoreType.DMA,
            pltpu.SemaphoreType.DMA,
        ],
    )(user_id, emb_user_t)


def _tc_dense_body(x_ref, rwx_ref, wx_ref, out_ref, ssq_ref):
    x = x_ref[...]
    ws = wx_ref[...] * rwx_ref[...]
    out_ref[...] = jnp.dot(x, ws, preferred_element_type=jnp.float32)
    ssq_ref[...] = jnp.sum(x * x, axis=1, keepdims=True)


def _tc_dense_partial(dense_profile, rms_weight, W, blk=1024):
    grid = _B // blk
    rwx = lax.slice(rms_weight, (_TOTAL - _D_DENSE,), (_TOTAL,)).reshape(
        _D_DENSE, 1)
    wx = lax.slice(W, (_TOTAL - _D_DENSE, 0), (_TOTAL, _OUT_D))
    return pl.pallas_call(
        _tc_dense_body,
        grid=(grid,),
        in_specs=[
            pl.BlockSpec((blk, _D_DENSE), lambda i: (i, 0)),
            pl.BlockSpec((_D_DENSE, 1), lambda i: (0, 0)),
            pl.BlockSpec((_D_DENSE, _OUT_D), lambda i: (0, 0)),
        ],
        out_specs=[
            pl.BlockSpec((blk, _OUT_D), lambda i: (i, 0)),
            pl.BlockSpec((blk, 1), lambda i: (i, 0)),
        ],
        out_shape=[
            jax.ShapeDtypeStruct((_B, _OUT_D), jnp.float32),
            jax.ShapeDtypeStruct((_B, 1), jnp.float32),
        ],
    )(dense_profile, rwx, wx)


_D_EMB = _D_USER + _D_COUNTRY + _D_DEVICE


def _lane_gather(tblt, idx2):
    ilo = jnp.bitwise_and(idx2, 127)
    ihi = jnp.right_shift(idx2, 7)
    out = jnp.zeros(idx2.shape, jnp.float32)
    for ci in range(8):
        part = jnp.take_along_axis(tblt[:, ci * 128:(ci + 1) * 128], ilo,
                                   axis=1)
        out = jnp.where(ihi == ci, part, out)
    return out


def _tc_combine_body(ut_ref, ci_ref, di_ref, ctbl_ref, dtbl_ref, part_ref,
                     ssqx_ref, rwe_ref, we_ref, b_ref, out_ref):
    ut = ut_ref[...]
    ct = _lane_gather(ctbl_ref[...], ci_ref[...])
    dt = _lane_gather(dtbl_ref[...], di_ref[...])
    part = part_ref[...]
    dnums_t = (((0,), (0,)), ((), ()))
    ones_u = jnp.ones((_D_USER, 1), jnp.float32)
    ones_s = jnp.ones((_D_COUNTRY, 1), jnp.float32)
    ssq = (lax.dot_general(ut * ut, ones_u, dnums_t,
                           preferred_element_type=jnp.float32)
           + lax.dot_general(ct * ct, ones_s, dnums_t,
                             preferred_element_type=jnp.float32)
           + lax.dot_general(dt * dt, ones_s, dnums_t,
                             preferred_element_type=jnp.float32)
           + ssqx_ref[...])
    scale = lax.rsqrt(ssq * (1.0 / _TOTAL) + _EPS)
    ws = we_ref[...] * rwe_ref[...]
    s0, s1 = _D_USER, _D_USER + _D_COUNTRY
    acc = lax.dot_general(ut, ws[0:s0], dnums_t,
                          preferred_element_type=jnp.float32)
    acc += lax.dot_general(ct, ws[s0:s1], dnums_t,
                           preferred_element_type=jnp.float32)
    acc += lax.dot_general(dt, ws[s1:_D_EMB], dnums_t,
                           preferred_element_type=jnp.float32)
    acc += part
    out_ref[...] = acc * scale + b_ref[...]


def _tc_combine(e_user_t, cidx_b, didx_b, ctblt, dtblt, partial, ssq_x,
                rms_weight, W, b, blk=1024):
    grid = _B // blk
    rwe = lax.slice(rms_weight, (0,), (_D_EMB,)).reshape(_D_EMB, 1)
    we = lax.slice(W, (0, 0), (_D_EMB, _OUT_D))
    b2 = b.reshape(1, _OUT_D)
    return pl.pallas_call(
        _tc_combine_body,
        grid=(grid,),
        in_specs=[
            pl.BlockSpec((_D_USER, blk), lambda i: (0, i)),
            pl.BlockSpec((_D_COUNTRY, blk), lambda i: (0, i)),
            pl.BlockSpec((_D_DEVICE, blk), lambda i: (0, i)),
            pl.BlockSpec((_D_COUNTRY, _VPAD), lambda i: (0, 0)),
            pl.BlockSpec((_D_DEVICE, _VPAD), lambda i: (0, 0)),
            pl.BlockSpec((blk, _OUT_D), lambda i: (i, 0)),
            pl.BlockSpec((blk, 1), lambda i: (i, 0)),
            pl.BlockSpec((_D_EMB, 1), lambda i: (0, 0)),
            pl.BlockSpec((_D_EMB, _OUT_D), lambda i: (0, 0)),
            pl.BlockSpec((1, _OUT_D), lambda i: (0, 0)),
        ],
        out_specs=pl.BlockSpec((blk, _OUT_D), lambda i: (i, 0)),
        out_shape=jax.ShapeDtypeStruct((_B, _OUT_D), jnp.float32),
    )(e_user_t, cidx_b, didx_b, ctblt, dtblt, partial, ssq_x, rwe, we, b2)


def kernel(user_id, country, device, dense_profile, emb_user, emb_country,
           emb_device, rms_weight, W, b):
    e_user_t = _sc_user_gather(user_id.astype(jnp.int32), emb_user.T)
    partial, ssq_x = _tc_dense_partial(dense_profile, rms_weight, W)
    cidx_b = jnp.broadcast_to(country.reshape(1, _B), (_D_COUNTRY, _B))
    didx_b = jnp.broadcast_to(device.reshape(1, _B), (_D_DEVICE, _B))
    ctblt = jnp.pad(emb_country.T, ((0, 0), (0, _VPAD - _V_SMALL)))
    dtblt = jnp.pad(emb_device.T, ((0, 0), (0, _VPAD - _V_SMALL)))
    return _tc_combine(e_user_t, cidx_b, didx_b, ctblt, dtblt, partial,
                       ssq_x, rms_weight, W, b)


# speedup vs baseline: 1.1024x; 1.0025x over previous
import functools

import jax
import jax.numpy as jnp
from jax import lax
from jax.experimental import pallas as pl
from jax.experimental.pallas import tpu as pltpu
from jax.experimental.pallas import tpu_sc as plsc

_B = 16384
_V_USER, _V_SMALL = 1000000, 1000
_D_USER, _D_COUNTRY, _D_DEVICE, _D_DENSE = 32, 16, 16, 128
_TOTAL = _D_USER + _D_COUNTRY + _D_DEVICE + _D_DENSE
_OUT_D = 128
_EPS = 1.1920928955078125e-07
_VPAD = 1024

_NC, _NS = 2, 16
_BPS = _B // _NS
_STAGE = 65536
_STAGE_TAIL = _V_USER - 15 * _STAGE


def _sc_user_body(uid_hbm, eu_t, out_u, uidx_v, dst_v, row_a, row_b,
                  sem, sem_st):
    c = lax.axis_index("c")
    s = lax.axis_index("s")

    def stage_copies(jl, row_sh):
        j = c * _NS + jl
        main = pltpu.make_async_copy(
            eu_t.at[pl.ds(j, 1), pl.ds(s * _STAGE, _STAGE)],
            row_sh.at[pl.ds(0, 1), pl.ds(s * _STAGE, _STAGE)],
            sem_st)
        tail = pltpu.make_async_copy(
            eu_t.at[pl.ds(j, 1), pl.ds(15 * _STAGE, _STAGE_TAIL)],
            row_sh.at[pl.ds(0, 1), pl.ds(15 * _STAGE, _STAGE_TAIL)],
            sem_st)
        return main, tail

    def stage_start(jl, row_sh):
        main, tail = stage_copies(jl, row_sh)

        @pl.when(s < 15)
        def _():
            main.start()

        @pl.when(s == 15)
        def _():
            tail.start()

    def stage_wait(jl, row_sh):
        main, tail = stage_copies(jl, row_sh)

        @pl.when(s < 15)
        def _():
            main.wait()

        @pl.when(s == 15)
        def _():
            tail.wait()

    stage_start(0, row_a)
    pltpu.sync_copy(uid_hbm.at[pl.ds(s * _BPS, _BPS)], uidx_v)
    stage_wait(0, row_a)
    plsc.subcore_barrier()

    def half_step(jl, cur_row, nxt_row):
        j = c * _NS + jl

        @pl.when(jl < _NS - 1)
        def _():
            stage_start(jl + 1, nxt_row)

        pltpu.async_copy(cur_row.at[0].at[uidx_v], dst_v.at[0], sem).wait()
        pltpu.sync_copy(dst_v, out_u.at[pl.ds(j, 1), pl.ds(s * _BPS, _BPS)])

        @pl.when(jl < _NS - 1)
        def _():
            stage_wait(jl + 1, nxt_row)

        plsc.subcore_barrier()

    def step(p, carry):
        half_step(2 * p, row_a, row_b)
        half_step(2 * p + 1, row_b, row_a)
        return carry

    lax.fori_loop(0, _NS // 2, step, 0)


def _sc_user_gather(user_id, emb_user_t):
    return pl.kernel(
        _sc_user_body,
        out_type=jax.ShapeDtypeStruct((_D_USER, _B), jnp.float32),
        mesh=plsc.VectorSubcoreMesh(core_axis_name="c", subcore_axis_name="s"),
        compiler_params=pltpu.CompilerParams(use_tc_tiling_on_sc=True),
        scratch_types=[
            pltpu.VMEM((_BPS,), jnp.int32),
            pltpu.VMEM((1, _BPS), jnp.float32),
            pltpu.VMEM_SHARED((1, _V_USER), jnp.float32),
            pltpu.VMEM_SHARED((1, _V_USER), jnp.float32),
            pltpu.SemaphoreType.DMA,
            pltpu.SemaphoreType.DMA,
        ],
    )(user_id, emb_user_t)


def _tc_dense_body(x_ref, rwx_ref, wx_ref, out_ref, ssq_ref):
    x = x_ref[...]
    ws = wx_ref[...] * rwx_ref[...]
    out_ref[...] = jnp.dot(x, ws, preferred_element_type=jnp.float32)
    ssq_ref[...] = jnp.sum(x * x, axis=1, keepdims=True)


def _tc_dense_partial(dense_profile, rms_weight, W, blk=2048):
    grid = _B // blk
    rwx = lax.slice(rms_weight, (_TOTAL - _D_DENSE,), (_TOTAL,)).reshape(
        _D_DENSE, 1)
    wx = lax.slice(W, (_TOTAL - _D_DENSE, 0), (_TOTAL, _OUT_D))
    return pl.pallas_call(
        _tc_dense_body,
        grid=(grid,),
        in_specs=[
            pl.BlockSpec((blk, _D_DENSE), lambda i: (i, 0)),
            pl.BlockSpec((_D_DENSE, 1), lambda i: (0, 0)),
            pl.BlockSpec((_D_DENSE, _OUT_D), lambda i: (0, 0)),
        ],
        out_specs=[
            pl.BlockSpec((blk, _OUT_D), lambda i: (i, 0)),
            pl.BlockSpec((blk, 1), lambda i: (i, 0)),
        ],
        out_shape=[
            jax.ShapeDtypeStruct((_B, _OUT_D), jnp.float32),
            jax.ShapeDtypeStruct((_B, 1), jnp.float32),
        ],
    )(dense_profile, rwx, wx)


_D_EMB = _D_USER + _D_COUNTRY + _D_DEVICE


def _lane_gather(tblt, idx2):
    ilo = jnp.bitwise_and(idx2, 127)
    ihi = jnp.right_shift(idx2, 7)
    out = jnp.zeros(idx2.shape, jnp.float32)
    for ci in range(8):
        part = jnp.take_along_axis(tblt[:, ci * 128:(ci + 1) * 128], ilo,
                                   axis=1)
        out = jnp.where(ihi == ci, part, out)
    return out


def _tc_combine_body(ut_ref, ci_ref, di_ref, ctbl_ref, dtbl_ref, part_ref,
                     ssqx_ref, rwe_ref, we_ref, b_ref, out_ref):
    ut = ut_ref[...]
    ct = _lane_gather(ctbl_ref[...], ci_ref[...])
    dt = _lane_gather(dtbl_ref[...], di_ref[...])
    part = part_ref[...]
    dnums_t = (((0,), (0,)), ((), ()))
    ones_u = jnp.ones((_D_USER, 1), jnp.float32)
    ones_s = jnp.ones((_D_COUNTRY, 1), jnp.float32)
    ssq = (lax.dot_general(ut * ut, ones_u, dnums_t,
                           preferred_element_type=jnp.float32)
           + lax.dot_general(ct * ct, ones_s, dnums_t,
                             preferred_element_type=jnp.float32)
           + lax.dot_general(dt * dt, ones_s, dnums_t,
                             preferred_element_type=jnp.float32)
           + ssqx_ref[...])
    scale = lax.rsqrt(ssq * (1.0 / _TOTAL) + _EPS)
    ws = we_ref[...] * rwe_ref[...]
    s0, s1 = _D_USER, _D_USER + _D_COUNTRY
    acc = lax.dot_general(ut, ws[0:s0], dnums_t,
                          preferred_element_type=jnp.float32)
    acc += lax.dot_general(ct, ws[s0:s1], dnums_t,
                           preferred_element_type=jnp.float32)
    acc += lax.dot_general(dt, ws[s1:_D_EMB], dnums_t,
                           preferred_element_type=jnp.float32)
    acc += part
    out_ref[...] = acc * scale + b_ref[...]


def _tc_combine(e_user_t, cidx_b, didx_b, ctblt, dtblt, partial, ssq_x,
                rms_weight, W, b, blk=2048):
    grid = _B // blk
    rwe = lax.slice(rms_weight, (0,), (_D_EMB,)).reshape(_D_EMB, 1)
    we = lax.slice(W, (0, 0), (_D_EMB, _OUT_D))
    b2 = b.reshape(1, _OUT_D)
    return pl.pallas_call(
        _tc_combine_body,
        grid=(grid,),
        in_specs=[
            pl.BlockSpec((_D_USER, blk), lambda i: (0, i)),
            pl.BlockSpec((_D_COUNTRY, blk), lambda i: (0, i)),
            pl.BlockSpec((_D_DEVICE, blk), lambda i: (0, i)),
            pl.BlockSpec((_D_COUNTRY, _VPAD), lambda i: (0, 0)),
            pl.BlockSpec((_D_DEVICE, _VPAD), lambda i: (0, 0)),
            pl.BlockSpec((blk, _OUT_D), lambda i: (i, 0)),
            pl.BlockSpec((blk, 1), lambda i: (i, 0)),
            pl.BlockSpec((_D_EMB, 1), lambda i: (0, 0)),
            pl.BlockSpec((_D_EMB, _OUT_D), lambda i: (0, 0)),
            pl.BlockSpec((1, _OUT_D), lambda i: (0, 0)),
        ],
        out_specs=pl.BlockSpec((blk, _OUT_D), lambda i: (i, 0)),
        out_shape=jax.ShapeDtypeStruct((_B, _OUT_D), jnp.float32),
    )(e_user_t, cidx_b, didx_b, ctblt, dtblt, partial, ssq_x, rwe, we, b2)


def kernel(user_id, country, device, dense_profile, emb_user, emb_country,
           emb_device, rms_weight, W, b):
    e_user_t = _sc_user_gather(user_id.astype(jnp.int32), emb_user.T)
    partial, ssq_x = _tc_dense_partial(dense_profile, rms_weight, W)
    cidx_b = jnp.broadcast_to(country.reshape(1, _B), (_D_COUNTRY, _B))
    didx_b = jnp.broadcast_to(device.reshape(1, _B), (_D_DEVICE, _B))
    ctblt = jnp.pad(emb_country.T, ((0, 0), (0, _VPAD - _V_SMALL)))
    dtblt = jnp.pad(emb_device.T, ((0, 0), (0, _VPAD - _V_SMALL)))
    return _tc_combine(e_user_t, cidx_b, didx_b, ctblt, dtblt, partial,
                       ssq_x, rms_weight, W, b)
